# Initial kernel scaffold; baseline (speedup 1.0000x reference)
import jax
import jax.numpy as jnp
from jax.experimental import pallas as pl


def kernel(embeddings, target):
    raise NotImplementedError("write your pallas kernel here")


# speedup vs baseline: 2.5135x; 2.5135x over previous
import jax
import jax.numpy as jnp
from jax.experimental import pallas as pl
from jax.experimental.pallas import tpu as pltpu

_MARGIN = 1.0
_B = 256
_BIG = 1e30
_CHUNK = 8


def _triplet_kernel(emb_ref, tcol_ref, trow_ref, loss_ref, cnt_ref):
    e = emb_ref[:]
    g = jnp.dot(e, e.T, preferred_element_type=jnp.float32)
    sq = jnp.sum(e * e, axis=1)
    dist = sq[:, None] + sq[None, :] - 2.0 * g

    lab_eq = tcol_ref[:] == trow_ref[:]
    row_i = jax.lax.broadcasted_iota(jnp.int32, (_B, _B), 0)
    col_i = jax.lax.broadcasted_iota(jnp.int32, (_B, _B), 1)
    eye = row_i == col_i
    pos_mask = lab_eq & jnp.logical_not(eye)
    neg_mask = jnp.logical_not(lab_eq)

    dp = jnp.where(pos_mask, dist + _MARGIN, -_BIG)
    dn = jnp.where(neg_mask, dist, _BIG)

    loss_sum = jnp.float32(0.0)
    for i in range(_B // _CHUNK):
        dpc = dp[i * _CHUNK:(i + 1) * _CHUNK, :]
        dnc = dn[i * _CHUNK:(i + 1) * _CHUNK, :]
        term = jnp.maximum(dpc[:, :, None] - dnc[:, None, :], 0.0)
        loss_sum = loss_sum + jnp.sum(term)

    pos_cnt = jnp.sum(pos_mask.astype(jnp.int32), axis=1)
    neg_cnt = jnp.sum(neg_mask.astype(jnp.int32), axis=1)
    num = jnp.sum(pos_cnt * neg_cnt)
    loss = jnp.where(num > 0,
                     loss_sum / jnp.maximum(num, 1).astype(jnp.float32),
                     0.0)
    loss_ref[:, :] = jnp.reshape(loss, (1, 1))
    cnt_ref[:, :] = jnp.reshape(num, (1, 1))


def kernel(embeddings, target):
    t32 = target.astype(jnp.int32)
    tcol = t32.reshape(_B, 1)
    trow = t32.reshape(1, _B)
    loss, cnt = pl.pallas_call(
        _triplet_kernel,
        out_shape=(
            jax.ShapeDtypeStruct((1, 1), jnp.float32),
            jax.ShapeDtypeStruct((1, 1), jnp.int32),
        ),
    )(embeddings, tcol, trow)
    return loss[0, 0], cnt[0, 0]
